# baseline (speedup 1.0000x reference)
import functools

import jax
import jax.numpy as jnp
from jax import lax
from jax.experimental import pallas as pl
from jax.experimental.pallas import tpu as pltpu
from jax.experimental.pallas import tpu_sc as plsc

V = 1000000
D = 64
NC = 2
NS = 16
NW = NC * NS
L = 16

_MESH = plsc.VectorSubcoreMesh(core_axis_name="c", subcore_axis_name="s")
_PARAMS = pltpu.CompilerParams(
    use_tc_tiling_on_sc=True,
    needs_layout_passes=False,
    disable_bounds_checks=True,
)


def _worker_id():
    return lax.axis_index("s") * NC + lax.axis_index("c")


N_BLK = V // 128
TAIL = V - N_BLK * 128
BLK_PER_W = (N_BLK + NW - 1) // NW
A_NBUF = 4
A_ITERS = (BLK_PER_W + A_NBUF - 1) // A_NBUF


@functools.partial(
    pl.kernel,
    out_type=jax.ShapeDtypeStruct((V // 2, 128), jnp.float32),
    mesh=_MESH,
    scratch_types=[
        pltpu.VMEM((64, 128), jnp.float32),
        pltpu.VMEM((64, 128), jnp.float32),
        pltpu.VMEM((64, 128), jnp.float32),
        pltpu.VMEM((64, 128), jnp.float32),
        pltpu.VMEM((64, 128), jnp.float32),
        pltpu.VMEM((64, 128), jnp.float32),
        pltpu.VMEM((64, 128), jnp.float32),
        pltpu.VMEM((64, 128), jnp.float32),
        pltpu.VMEM((64, 64), jnp.float32),
        pltpu.SemaphoreType.DMA,
        pltpu.SemaphoreType.DMA,
        pltpu.SemaphoreType.DMA,
        pltpu.SemaphoreType.DMA,
        pltpu.SemaphoreType.DMA,
        pltpu.SemaphoreType.DMA,
        pltpu.SemaphoreType.DMA,
        pltpu.SemaphoreType.DMA,
    ],
    compiler_params=_PARAMS,
)
def _transpose_table(wt_ref, out_ref, n0, n1, n2, n3, t0, t1, t2, t3, n64_v,
                     rs0, rs1, rs2, rs3, ws0, ws1, ws2, ws3):
    wid = _worker_id()
    iota = lax.iota(jnp.int32, L)
    rvecs = [iota + (16 * gi) for gi in range(4)]
    base = wid * BLK_PER_W
    kend = jnp.minimum(base + BLK_PER_W, N_BLK)
    bufs = ((n0, t0, rs0, ws0), (n1, t1, rs1, ws1),
            (n2, t2, rs2, ws2), (n3, t3, rs3, ws3))

    def src_at(k):
        return wt_ref.at[:, pl.ds(pl.multiple_of(k * 128, 128), 128)]

    diag = [(iota + k) & 15 for k in range(16)]

    def transpose_block(n_v, t_v):
        def jb_body(jbi, _):
            jb = jbi * 16
            for gi in range(4):
                rvec = rvecs[gi]
                for k in range(16):
                    jv = jb + diag[k]
                    v = plsc.load_gather(n_v, [rvec, jv])
                    cv = (jv & 1) * 64 + rvec
                    plsc.store_scatter(t_v, [
                        lax.shift_right_logical(jv, 1), cv], v)
            return ()

        lax.fori_loop(0, 8, jb_body, (), unroll=False)

    for b in range(A_NBUF):
        k = base + b

        @pl.when(k < kend)
        def _(k=k, b=b):
            pltpu.async_copy(src_at(k), bufs[b][0], bufs[b][2])

    def quad_body(g, _):
        for b in range(A_NBUF):
            n_v, t_v, rsem, wsem = bufs[b]
            k = base + A_NBUF * g + b

            @pl.when(k < kend)
            def _(k=k, n_v=n_v, t_v=t_v, rsem=rsem, wsem=wsem):
                pltpu.make_async_copy(src_at(k), n_v, rsem).wait()

                @pl.when(g > 0)
                def _():
                    pltpu.make_async_copy(
                        t_v, out_ref.at[pl.ds(k * 64, 64)], wsem
                    ).wait()

                transpose_block(n_v, t_v)
                pltpu.async_copy(t_v, out_ref.at[pl.ds(k * 64, 64)], wsem)

                @pl.when(k + A_NBUF < kend)
                def _():
                    pltpu.async_copy(src_at(k + A_NBUF), n_v, rsem)

        return ()

    lax.fori_loop(0, A_ITERS, quad_body, (), unroll=False)

    for b in range(A_NBUF):
        @pl.when(base + b < kend)
        def _(b=b):
            pltpu.make_async_copy(
                bufs[b][1], out_ref.at[pl.ds(0, 64)], bufs[b][3]
            ).wait()

    @pl.when(wid == 0)
    def _():
        pltpu.sync_copy(wt_ref.at[:, pl.ds(N_BLK * 128, TAIL)], n64_v)

        def row_body(rp, _):
            for h in range(2):
                cvec = jnp.zeros((L,), jnp.int32) + (2 * rp + h)
                for gi in range(4):
                    v = plsc.load_gather(n64_v, [rvecs[gi], cvec])
                    t0[rp, pl.ds(64 * h + 16 * gi, L)] = v
            return ()

        lax.fori_loop(0, TAIL // 2, row_body, (), unroll=False)
        pltpu.sync_copy(
            t0.at[pl.ds(0, TAIL // 2)], out_ref.at[pl.ds(N_BLK * 64, TAIL // 2)]
        )


B_TOK = 16384
S_LEN = 50
N_ITEMS = S_LEN * (B_TOK // 128)
ITEM_PER_W = N_ITEMS // NW
B_PAIRS = ITEM_PER_W // 2


@functools.partial(
    pl.kernel,
    out_type=jax.ShapeDtypeStruct((S_LEN, D, B_TOK), jnp.float32),
    mesh=_MESH,
    scratch_types=[
        pltpu.VMEM((128,), jnp.int32),
        pltpu.VMEM((128,), jnp.int32),
        pltpu.VMEM((128,), jnp.int32),
        pltpu.VMEM((128,), jnp.int32),
        pltpu.VMEM((128,), jnp.int32),
        pltpu.VMEM((128,), jnp.int32),
        pltpu.VMEM((128, 128), jnp.float32),
        pltpu.VMEM((128, 128), jnp.float32),
        pltpu.VMEM((D, 128), jnp.float32),
        pltpu.VMEM((D, 128), jnp.float32),
        pltpu.SemaphoreType.DMA,
        pltpu.SemaphoreType.DMA,
        pltpu.SemaphoreType.DMA,
        pltpu.SemaphoreType.DMA,
        pltpu.SemaphoreType.DMA,
        pltpu.SemaphoreType.DMA,
    ],
    compiler_params=_PARAMS,
)
def _gather_embed(tab_ref, idx_ref, out_ref,
                  i0, i1, p0, p1, q0, q1, g0, g1, o0, o1,
                  is0, is1, gs0, gs1, ws0, ws1):
    wid = _worker_id()
    iota = lax.iota(jnp.int32, L)
    rvecs = [iota + (16 * ci) for ci in range(8)]
    base = wid * ITEM_PER_W
    bufs = (
        (i0, p0, q0, g0, o0, is0, gs0, ws0),
        (i1, p1, q1, g1, o1, is1, gs1, ws1),
    )

    def idx_src(item):
        s = item // 128
        b0 = (item % 128) * 128
        return idx_ref.at[s, pl.ds(b0, 128)]

    def out_dst(item):
        s = item // 128
        b0 = (item % 128) * 128
        return out_ref.at[s, :, pl.ds(b0, 128)]

    def prep(idx_v, p_v, par_v):
        for ci in range(8):
            sl = pl.ds(16 * ci, L)
            t = idx_v[sl]
            p_v[sl] = lax.shift_right_logical(t, 1)
            par_v[sl] = (t & 1) * 64

    diag = [(iota + k) & 15 for k in range(16)]

    def transpose_sel(g_v, par_v, o_v):
        def tb_body(tbi, _):
            tb = tbi * 16
            rvec = iota + tb
            par16 = par_v[pl.ds(tb, L)]
            for ebi in range(4):
                for k in range(16):
                    ce = diag[k] + (ebi * 16)
                    v = plsc.load_gather(g_v, [rvec, par16 + ce])
                    plsc.store_scatter(o_v, [ce, rvec], v)
            return ()

        lax.fori_loop(0, 8, tb_body, (), unroll=False)

    for b in range(2):
        item = base + b
        idx_v, p_v, par_v, g_v, o_v, isem, gsem, wsem = bufs[b]
        pltpu.async_copy(idx_src(item), idx_v, isem).wait()
        prep(idx_v, p_v, par_v)
        pltpu.async_copy(tab_ref.at[p_v], g_v, gsem)

    def pair_body(g, _):
        for b in range(2):
            idx_v, p_v, par_v, g_v, o_v, isem, gsem, wsem = bufs[b]
            item = base + 2 * g + b

            pltpu.make_async_copy(tab_ref.at[p_v], g_v, gsem).wait()

            @pl.when(g > 0)
            def _(o_v=o_v, wsem=wsem, item=item):
                pltpu.make_async_copy(o_v, out_dst(item), wsem).wait()

            @pl.when(g + 1 < B_PAIRS)
            def _(idx_v=idx_v, isem=isem, item=item):
                pltpu.async_copy(idx_src(item + 2), idx_v, isem)

            transpose_sel(g_v, par_v, o_v)
            pltpu.async_copy(o_v, out_dst(item), wsem)

            @pl.when(g + 1 < B_PAIRS)
            def _(idx_v=idx_v, p_v=p_v, par_v=par_v, g_v=g_v,
                  isem=isem, gsem=gsem):
                pltpu.make_async_copy(idx_src(base), idx_v, isem).wait()
                prep(idx_v, p_v, par_v)
                pltpu.async_copy(tab_ref.at[p_v], g_v, gsem)

        return ()

    lax.fori_loop(0, B_PAIRS, pair_body, (), unroll=False)

    for b in range(2):
        o_v, wsem = bufs[b][4], bufs[b][7]
        pltpu.make_async_copy(o_v, out_dst(base), wsem).wait()


def kernel(token_ids, embed_weights):
    wt_t = embed_weights.T
    idx_t = token_ids.astype(jnp.int32).T
    tab = _transpose_table(wt_t)
    out_t = _gather_embed(tab, idx_t)
    return out_t.transpose(2, 0, 1)


# speedup vs baseline: 1.1416x; 1.1416x over previous
import functools

import jax
import jax.numpy as jnp
from jax import lax
from jax.experimental import pallas as pl
from jax.experimental.pallas import tpu as pltpu
from jax.experimental.pallas import tpu_sc as plsc

V = 1000000
D = 64
NC = 2
NS = 16
NW = NC * NS
L = 16

_MESH = plsc.VectorSubcoreMesh(core_axis_name="c", subcore_axis_name="s")
_PARAMS = pltpu.CompilerParams(
    use_tc_tiling_on_sc=True,
    needs_layout_passes=False,
    disable_bounds_checks=True,
)


def _worker_id():
    return lax.axis_index("s") * NC + lax.axis_index("c")


A_W = 256
N_BLK = V // A_W
TAIL = V - N_BLK * A_W
BLK_PER_W = (N_BLK + NW - 1) // NW
A_PAIRS = (BLK_PER_W + 1) // 2


@functools.partial(
    pl.kernel,
    out_type=jax.ShapeDtypeStruct((V // 2, 128), jnp.float32),
    mesh=_MESH,
    scratch_types=[
        pltpu.VMEM((64, A_W), jnp.float32),
        pltpu.VMEM((64, A_W), jnp.float32),
        pltpu.VMEM((A_W // 2, 128), jnp.float32),
        pltpu.VMEM((A_W // 2, 128), jnp.float32),
        pltpu.VMEM((64, 64), jnp.float32),
        pltpu.SemaphoreType.DMA,
        pltpu.SemaphoreType.DMA,
        pltpu.SemaphoreType.DMA,
        pltpu.SemaphoreType.DMA,
    ],
    compiler_params=_PARAMS,
)
def _transpose_table(wt_ref, out_ref, n0, n1, t0, t1, n64_v,
                     rs0, rs1, ws0, ws1):
    wid = _worker_id()
    iota = lax.iota(jnp.int32, L)
    rvecs = [iota + (16 * gi) for gi in range(4)]
    base = wid * BLK_PER_W
    kend = jnp.minimum(base + BLK_PER_W, N_BLK)
    bufs = ((n0, t0, rs0, ws0), (n1, t1, rs1, ws1))

    def src_at(k):
        return wt_ref.at[:, pl.ds(pl.multiple_of(k * A_W, A_W), A_W)]

    diag = [(iota + k) & 15 for k in range(16)]

    def transpose_block(n_v, t_v):
        def jb_body(jbi, _):
            jb = jbi * 16
            for gi in range(4):
                rvec = rvecs[gi]
                for k in range(16):
                    jv = jb + diag[k]
                    v = plsc.load_gather(n_v, [rvec, jv])
                    cv = (jv & 1) * 64 + rvec
                    plsc.store_scatter(t_v, [
                        lax.shift_right_logical(jv, 1), cv], v)
            return ()

        lax.fori_loop(0, A_W // 16, jb_body, (), unroll=False)

    for b in range(2):
        k = base + b

        @pl.when(k < kend)
        def _(k=k, b=b):
            pltpu.async_copy(src_at(k), bufs[b][0], bufs[b][2])

    def pair_body(g, _):
        for b in range(2):
            n_v, t_v, rsem, wsem = bufs[b]
            k = base + 2 * g + b

            @pl.when(k < kend)
            def _(k=k, n_v=n_v, t_v=t_v, rsem=rsem, wsem=wsem):
                pltpu.make_async_copy(src_at(k), n_v, rsem).wait()

                @pl.when(g > 0)
                def _():
                    pltpu.make_async_copy(
                        t_v, out_ref.at[pl.ds(k * 128, 128)], wsem
                    ).wait()

                transpose_block(n_v, t_v)
                pltpu.async_copy(t_v, out_ref.at[pl.ds(k * 128, 128)], wsem)

                @pl.when(k + 2 < kend)
                def _():
                    pltpu.async_copy(src_at(k + 2), n_v, rsem)

        return ()

    lax.fori_loop(0, A_PAIRS, pair_body, (), unroll=False)

    for b in range(2):
        @pl.when(base + b < kend)
        def _(b=b):
            pltpu.make_async_copy(
                bufs[b][1], out_ref.at[pl.ds(0, 128)], bufs[b][3]
            ).wait()

    @pl.when(wid == 0)
    def _():
        pltpu.sync_copy(wt_ref.at[:, pl.ds(N_BLK * A_W, TAIL)], n64_v)

        def row_body(rp, _):
            for h in range(2):
                cvec = jnp.zeros((L,), jnp.int32) + (2 * rp + h)
                for gi in range(4):
                    v = plsc.load_gather(n64_v, [rvecs[gi], cvec])
                    t0[rp, pl.ds(64 * h + 16 * gi, L)] = v
            return ()

        lax.fori_loop(0, TAIL // 2, row_body, (), unroll=False)
        pltpu.sync_copy(
            t0.at[pl.ds(0, TAIL // 2)], out_ref.at[pl.ds(N_BLK * (A_W // 2), TAIL // 2)]
        )


B_TOK = 16384
S_LEN = 50
N_ITEMS = S_LEN * (B_TOK // 128)
ITEM_PER_W = N_ITEMS // NW
B_PAIRS = ITEM_PER_W // 2


@functools.partial(
    pl.kernel,
    out_type=jax.ShapeDtypeStruct((S_LEN, D, B_TOK), jnp.float32),
    mesh=_MESH,
    scratch_types=[
        pltpu.VMEM((128,), jnp.int32),
        pltpu.VMEM((128,), jnp.int32),
        pltpu.VMEM((128,), jnp.int32),
        pltpu.VMEM((128,), jnp.int32),
        pltpu.VMEM((128,), jnp.int32),
        pltpu.VMEM((128,), jnp.int32),
        pltpu.VMEM((128, 128), jnp.float32),
        pltpu.VMEM((128, 128), jnp.float32),
        pltpu.VMEM((D, 128), jnp.float32),
        pltpu.VMEM((D, 128), jnp.float32),
        pltpu.SemaphoreType.DMA,
        pltpu.SemaphoreType.DMA,
        pltpu.SemaphoreType.DMA,
        pltpu.SemaphoreType.DMA,
        pltpu.SemaphoreType.DMA,
        pltpu.SemaphoreType.DMA,
    ],
    compiler_params=_PARAMS,
)
def _gather_embed(tab_ref, idx_ref, out_ref,
                  i0, i1, p0, p1, q0, q1, g0, g1, o0, o1,
                  is0, is1, gs0, gs1, ws0, ws1):
    wid = _worker_id()
    iota = lax.iota(jnp.int32, L)
    rvecs = [iota + (16 * ci) for ci in range(8)]
    base = wid * ITEM_PER_W
    bufs = (
        (i0, p0, q0, g0, o0, is0, gs0, ws0),
        (i1, p1, q1, g1, o1, is1, gs1, ws1),
    )

    def idx_src(item):
        s = item // 128
        b0 = (item % 128) * 128
        return idx_ref.at[s, pl.ds(b0, 128)]

    def out_dst(item):
        s = item // 128
        b0 = (item % 128) * 128
        return out_ref.at[s, :, pl.ds(b0, 128)]

    def prep(idx_v, p_v, par_v):
        for ci in range(8):
            sl = pl.ds(16 * ci, L)
            t = idx_v[sl]
            p_v[sl] = lax.shift_right_logical(t, 1)
            par_v[sl] = (t & 1) * 64

    diag = [(iota + k) & 15 for k in range(16)]

    def transpose_sel(g_v, par_v, o_v):
        def tb_body(tbi, _):
            tb = tbi * 16
            rvec = iota + tb
            par16 = par_v[pl.ds(tb, L)]
            for ebi in range(4):
                for k in range(16):
                    ce = diag[k] + (ebi * 16)
                    v = plsc.load_gather(g_v, [rvec, par16 + ce])
                    plsc.store_scatter(o_v, [ce, rvec], v)
            return ()

        lax.fori_loop(0, 8, tb_body, (), unroll=False)

    for b in range(2):
        item = base + b
        idx_v, p_v, par_v, g_v, o_v, isem, gsem, wsem = bufs[b]
        pltpu.async_copy(idx_src(item), idx_v, isem).wait()
        prep(idx_v, p_v, par_v)
        pltpu.async_copy(tab_ref.at[p_v], g_v, gsem)

    def pair_body(g, _):
        for b in range(2):
            idx_v, p_v, par_v, g_v, o_v, isem, gsem, wsem = bufs[b]
            item = base + 2 * g + b

            pltpu.make_async_copy(tab_ref.at[p_v], g_v, gsem).wait()

            @pl.when(g > 0)
            def _(o_v=o_v, wsem=wsem, item=item):
                pltpu.make_async_copy(o_v, out_dst(item), wsem).wait()

            @pl.when(g + 1 < B_PAIRS)
            def _(idx_v=idx_v, isem=isem, item=item):
                pltpu.async_copy(idx_src(item + 2), idx_v, isem)

            transpose_sel(g_v, par_v, o_v)
            pltpu.async_copy(o_v, out_dst(item), wsem)

            @pl.when(g + 1 < B_PAIRS)
            def _(idx_v=idx_v, p_v=p_v, par_v=par_v, g_v=g_v,
                  isem=isem, gsem=gsem):
                pltpu.make_async_copy(idx_src(base), idx_v, isem).wait()
                prep(idx_v, p_v, par_v)
                pltpu.async_copy(tab_ref.at[p_v], g_v, gsem)

        return ()

    lax.fori_loop(0, B_PAIRS, pair_body, (), unroll=False)

    for b in range(2):
        o_v, wsem = bufs[b][4], bufs[b][7]
        pltpu.make_async_copy(o_v, out_dst(base), wsem).wait()


def kernel(token_ids, embed_weights):
    wt_t = embed_weights.T
    idx_t = token_ids.astype(jnp.int32).T
    tab = _transpose_table(wt_t)
    out_t = _gather_embed(tab, idx_t)
    return out_t.transpose(2, 0, 1)
